# baseline (speedup 1.0000x reference)
import jax
import jax.numpy as jnp
from jax.experimental import pallas as pl
from jax.experimental.pallas import tpu as pltpu

B = 4
C = 4
N1 = 16384
SPLIT2 = 2


def _copy_body(src_ref, out_ref):
    out_ref[...] = src_ref[...]


def kernel(source, target, T_prev):
    del target, T_prev
    out = pl.pallas_call(
        _copy_body,
        out_shape=jax.ShapeDtypeStruct((B, C, N1), jnp.float32),
        grid=(1,),
        in_specs=[pl.BlockSpec((B, C, N1), lambda i: (0, 0, 0))],
        out_specs=pl.BlockSpec((B, C, N1), lambda i: (0, 0, 0)),
    )(source)
    return jnp.transpose(out, (0, 2, 1))


# speedup vs baseline: 1.0608x; 1.0608x over previous
import jax
import jax.numpy as jnp
from jax.experimental import pallas as pl

B = 4
C = 4
N1 = 16384
SPLIT = 2


def _copy_body(src_ref, out_ref):
    out_ref[...] = src_ref[...]


def kernel(source, target, T_prev):
    del target, T_prev
    out = pl.pallas_call(
        _copy_body,
        out_shape=jax.ShapeDtypeStruct((B, C, N1), jnp.float32),
        grid=(SPLIT,),
        in_specs=[pl.BlockSpec((B // SPLIT, C, N1), lambda i: (i, 0, 0))],
        out_specs=pl.BlockSpec((B // SPLIT, C, N1), lambda i: (i, 0, 0)),
    )(source)
    return jnp.transpose(out, (0, 2, 1))


# speedup vs baseline: 1.0909x; 1.0284x over previous
import jax
import jax.numpy as jnp
from jax.experimental import pallas as pl
from jax.experimental.pallas import tpu as pltpu

B = 4
C = 4
N1 = 16384


def _copy_body(src_hbm, out_hbm, buf, rsem, wsem):
    reads = [
        pltpu.make_async_copy(src_hbm.at[b], buf.at[b], rsem.at[b])
        for b in range(B)
    ]
    writes = [
        pltpu.make_async_copy(buf.at[b], out_hbm.at[b], wsem.at[b])
        for b in range(B)
    ]
    for r in reads:
        r.start()
    for b in range(B):
        reads[b].wait()
        writes[b].start()
    for w in writes:
        w.wait()


def kernel(source, target, T_prev):
    del target, T_prev
    out = pl.pallas_call(
        _copy_body,
        out_shape=jax.ShapeDtypeStruct((B, C, N1), jnp.float32),
        in_specs=[pl.BlockSpec(memory_space=pltpu.MemorySpace.HBM)],
        out_specs=pl.BlockSpec(memory_space=pltpu.MemorySpace.HBM),
        scratch_shapes=[
            pltpu.VMEM((B, C, N1), jnp.float32),
            pltpu.SemaphoreType.DMA((B,)),
            pltpu.SemaphoreType.DMA((B,)),
        ],
    )(source)
    return jnp.transpose(out, (0, 2, 1))
